# Initial kernel scaffold; baseline (speedup 1.0000x reference)
import jax
import jax.numpy as jnp
from jax.experimental import pallas as pl


def kernel(x, codebook, logits, lmbda):
    raise NotImplementedError("write your pallas kernel here")


# speedup vs baseline: 1.0722x; 1.0722x over previous
import math

import jax
import jax.numpy as jnp
from jax import lax
from jax.experimental import pallas as pl
from jax.experimental.pallas import tpu as pltpu

NCB = 16
CB_DIM = 4
CB_SIZE = 1024
TN = 1024


def _body(inv_l_ref, x_ref, cb_ref, cbt_ref, logits_ref, xh_ref, rate_ref):
    b = pl.program_id(0)
    i = pl.program_id(1)
    inv_l = inv_l_ref[0]

    logits = logits_ref[0]
    m = jnp.max(logits, axis=-1, keepdims=True)
    lse = jnp.log(jnp.sum(jnp.exp(logits - m), axis=-1, keepdims=True)) + m
    l2pmf = (logits - lse) * jnp.float32(-1.0 / math.log(2.0))

    cbt = cbt_ref[0]
    cb2 = jnp.sum(cbt * cbt, axis=0, keepdims=True)
    const = cb2 + l2pmf * inv_l

    xb = x_ref[0]
    dots = lax.dot_general(xb, cbt, (((1,), (0,)), ((), ())),
                           preferred_element_type=jnp.float32)
    dist = const - 2.0 * dots
    idx = jnp.argmin(dist, axis=-1)
    oh = (lax.broadcasted_iota(jnp.int32, (TN, CB_SIZE), 1)
          == idx[:, None]).astype(jnp.float32)
    cb = cb_ref[0]
    xh_ref[0] = jnp.dot(oh, cb, preferred_element_type=jnp.float32)

    @pl.when(jnp.logical_and(b == 0, i == 0))
    def _():
        rate_ref[0] = jnp.float32(0.0)

    rate_ref[0] += jnp.sum(oh * l2pmf)


def kernel(x, codebook, logits, lmbda):
    shape = x.shape
    xf = x.reshape(-1, NCB, CB_DIM)
    n = xf.shape[0]
    xt = xf.transpose(1, 0, 2)
    cbt = codebook.transpose(0, 2, 1)
    inv_l = (jnp.float32(1.0) / jnp.asarray(lmbda, jnp.float32)).reshape(1)

    xh_t, rate = pl.pallas_call(
        _body,
        grid=(NCB, n // TN),
        in_specs=[
            pl.BlockSpec(memory_space=pltpu.SMEM),
            pl.BlockSpec((1, TN, CB_DIM), lambda b, i: (b, i, 0)),
            pl.BlockSpec((1, CB_SIZE, CB_DIM), lambda b, i: (b, 0, 0)),
            pl.BlockSpec((1, CB_DIM, CB_SIZE), lambda b, i: (b, 0, 0)),
            pl.BlockSpec((1, 1, CB_SIZE), lambda b, i: (b, 0, 0)),
        ],
        out_specs=[
            pl.BlockSpec((1, TN, CB_DIM), lambda b, i: (b, i, 0)),
            pl.BlockSpec(memory_space=pltpu.SMEM),
        ],
        out_shape=[
            jax.ShapeDtypeStruct((NCB, n, CB_DIM), jnp.float32),
            jax.ShapeDtypeStruct((1,), jnp.float32),
        ],
    )(inv_l, xt, codebook, cbt, logits.reshape(NCB, 1, CB_SIZE))

    x_hat = xh_t.transpose(1, 0, 2).reshape(shape)
    zero = jnp.zeros((1,), dtype=jnp.float32)
    return (x_hat, rate.reshape(()), jnp.zeros((), jnp.float32), zero, zero)


# speedup vs baseline: 1.1542x; 1.0765x over previous
import math

import jax
import jax.numpy as jnp
from jax import lax
from jax.experimental import pallas as pl
from jax.experimental.pallas import tpu as pltpu

NCB = 16
CB_DIM = 4
CB_SIZE = 1024
TN = 2048


def _body(inv_l_ref, x_ref, cba_ref, cbt_ref, logits_ref, xh_ref, rate_ref):
    b = pl.program_id(0)
    i = pl.program_id(1)
    inv_l = inv_l_ref[0]

    logits = logits_ref[0]
    m = jnp.max(logits, axis=-1, keepdims=True)
    lse = jnp.log(jnp.sum(jnp.exp(logits - m), axis=-1, keepdims=True)) + m
    l2pmf = (logits - lse) * jnp.float32(-1.0 / math.log(2.0))

    cbt = cbt_ref[0]
    cb2 = jnp.sum(cbt * cbt, axis=0, keepdims=True)
    const = cb2 + l2pmf * inv_l

    xb = x_ref[0]
    dots = lax.dot_general(xb, cbt, (((1,), (0,)), ((), ())),
                           preferred_element_type=jnp.float32)
    dist = const - 2.0 * dots
    idx = jnp.argmin(dist, axis=-1)
    oh = (lax.broadcasted_iota(jnp.int32, (TN, CB_SIZE), 1)
          == idx[:, None]).astype(jnp.float32)
    res = jnp.dot(oh, cba_ref[0], preferred_element_type=jnp.float32)
    xh_ref[0] = res[:, :CB_DIM]

    @pl.when(jnp.logical_and(b == 0, i == 0))
    def _():
        rate_ref[0] = jnp.float32(0.0)

    rate_ref[0] += jnp.sum(res[:, CB_DIM])


def kernel(x, codebook, logits, lmbda):
    shape = x.shape
    xf = x.reshape(-1, NCB, CB_DIM)
    n = xf.shape[0]
    xt = xf.transpose(1, 0, 2)
    cbt = codebook.transpose(0, 2, 1)
    l2pmf = jax.nn.log_softmax(logits, axis=-1) * jnp.float32(-1.0 / math.log(2.0))
    cba = jnp.concatenate(
        [codebook, l2pmf[..., None],
         jnp.zeros((NCB, CB_SIZE, 3), jnp.float32)], axis=-1)
    inv_l = (jnp.float32(1.0) / jnp.asarray(lmbda, jnp.float32)).reshape(1)

    xh_t, rate = pl.pallas_call(
        _body,
        grid=(NCB, n // TN),
        in_specs=[
            pl.BlockSpec(memory_space=pltpu.SMEM),
            pl.BlockSpec((1, TN, CB_DIM), lambda b, i: (b, i, 0)),
            pl.BlockSpec((1, CB_SIZE, 2 * CB_DIM), lambda b, i: (b, 0, 0)),
            pl.BlockSpec((1, CB_DIM, CB_SIZE), lambda b, i: (b, 0, 0)),
            pl.BlockSpec((1, 1, CB_SIZE), lambda b, i: (b, 0, 0)),
        ],
        out_specs=[
            pl.BlockSpec((1, TN, CB_DIM), lambda b, i: (b, i, 0)),
            pl.BlockSpec(memory_space=pltpu.SMEM),
        ],
        out_shape=[
            jax.ShapeDtypeStruct((NCB, n, CB_DIM), jnp.float32),
            jax.ShapeDtypeStruct((1,), jnp.float32),
        ],
    )(inv_l, xt, cba, cbt, logits.reshape(NCB, 1, CB_SIZE))

    x_hat = xh_t.transpose(1, 0, 2).reshape(shape)
    zero = jnp.zeros((1,), dtype=jnp.float32)
    return (x_hat, rate.reshape(()), jnp.zeros((), jnp.float32), zero, zero)
